# Initial kernel scaffold; baseline (speedup 1.0000x reference)
import jax
import jax.numpy as jnp
from jax.experimental import pallas as pl


def kernel(x, H, W1_1, W2_1, W3_1, a_1, a2_1, q_1, W1_2, W2_2, W3_2, a_2, a2_2, q_2):
    raise NotImplementedError("write your pallas kernel here")


# speedup vs baseline: 1.4247x; 1.4247x over previous
import functools

import jax
import jax.numpy as jnp
from jax.experimental import pallas as pl
from jax.experimental.pallas import tpu as pltpu

_ALPHA = 0.2
_NEG = -9e15


def _dot(a, b):
    return jnp.dot(a, b, preferred_element_type=jnp.float32)


def _lrelu(x):
    return jnp.where(x > 0, x, _ALPHA * x)


def _prep_kernel(x_ref, w1_ref, w2_ref, a_ref, a2_ref, q_ref,
                 xt_ref, ln_ref, tn_ref, *, d):
    xb = x_ref[...]
    x4 = _dot(xb, w2_ref[...])
    xt_ref[...] = _dot(xb, w1_ref[...])
    sq = _dot(q_ref[...], a_ref[:d, :])
    s = _dot(x4, a_ref[d:, :]) + sq[0, 0]
    ln_ref[...] = _lrelu(s)
    tn_ref[...] = _dot(x4, a2_ref[:d, :])


def _prep(x2, w1, w2, a, a2, q, nb):
    n, d = x2.shape
    grid = n // nb
    return pl.pallas_call(
        functools.partial(_prep_kernel, d=d),
        grid=(grid,),
        in_specs=[
            pl.BlockSpec((nb, d), lambda i: (i, 0)),
            pl.BlockSpec((d, d), lambda i: (0, 0)),
            pl.BlockSpec((d, d), lambda i: (0, 0)),
            pl.BlockSpec((2 * d, 1), lambda i: (0, 0)),
            pl.BlockSpec((2 * d, 1), lambda i: (0, 0)),
            pl.BlockSpec((1, d), lambda i: (0, 0)),
        ],
        out_specs=[
            pl.BlockSpec((nb, d), lambda i: (i, 0)),
            pl.BlockSpec((nb, 1), lambda i: (i, 0)),
            pl.BlockSpec((nb, 1), lambda i: (i, 0)),
        ],
        out_shape=[
            jax.ShapeDtypeStruct((n, d), jnp.float32),
            jax.ShapeDtypeStruct((n, 1), jnp.float32),
            jax.ShapeDtypeStruct((n, 1), jnp.float32),
        ],
    )(x2, w1, w2, a, a2, q)


def _edge_kernel(h_ref, xt_ref, ln_ref, w3_ref, a2_ref,
                 edge_ref, te_ref, ht_ref, wxt_scr, w_scr, *, d, with_ht):
    i = pl.program_id(0)

    @pl.when(i == 0)
    def _():
        ln = ln_ref[...]
        m = jnp.max(ln)
        w = jnp.exp(ln - m)
        w_scr[...] = w
        wxt_scr[...] = xt_ref[...] * w

    hb = h_ref[...]
    if with_ht:
        ht_ref[...] = hb.T
    num = _dot(hb, wxt_scr[...])
    z = _dot(hb, w_scr[...])
    edge = num / z
    edge_ref[...] = edge
    e4 = _dot(edge, w3_ref[...])
    te_ref[...] = jax.lax.dot_general(
        a2_ref[d:, :], e4, (((0,), (1,)), ((), ())),
        preferred_element_type=jnp.float32)


def _edge_kernel_noht(h_ref, xt_ref, ln_ref, w3_ref, a2_ref,
                      edge_ref, te_ref, wxt_scr, w_scr, *, d):
    _edge_kernel(h_ref, xt_ref, ln_ref, w3_ref, a2_ref,
                 edge_ref, te_ref, None, wxt_scr, w_scr, d=d, with_ht=False)


def _edge(h2, xt, ln, w3, a2, eb, with_ht):
    e, n = h2.shape
    d = xt.shape[1]
    grid = pl.cdiv(e, eb)
    e_pad = grid * eb
    out_specs = [
        pl.BlockSpec((eb, d), lambda i: (i, 0)),
        pl.BlockSpec((1, eb), lambda i: (0, i)),
    ]
    out_shape = [
        jax.ShapeDtypeStruct((e, d), jnp.float32),
        jax.ShapeDtypeStruct((1, e_pad), jnp.float32),
    ]
    if with_ht:
        body = functools.partial(_edge_kernel, d=d, with_ht=True)
        out_specs.append(pl.BlockSpec((n, eb), lambda i: (0, i)))
        out_shape.append(jax.ShapeDtypeStruct((n, e_pad), jnp.float32))
    else:
        body = functools.partial(_edge_kernel_noht, d=d)
    return pl.pallas_call(
        body,
        grid=(grid,),
        in_specs=[
            pl.BlockSpec((eb, n), lambda i: (i, 0)),
            pl.BlockSpec((n, d), lambda i: (0, 0)),
            pl.BlockSpec((n, 1), lambda i: (0, 0)),
            pl.BlockSpec((d, d), lambda i: (0, 0)),
            pl.BlockSpec((2 * d, 1), lambda i: (0, 0)),
        ],
        out_specs=out_specs,
        out_shape=out_shape,
        scratch_shapes=[
            pltpu.VMEM((n, d), jnp.float32),
            pltpu.VMEM((n, 1), jnp.float32),
        ],
    )(h2, xt, ln, w3, a2)


def _node_body(ht_ref, te_ref, tn_ref, edge_ref, xin_ref, *, e):
    ht = ht_ref[:, :e]
    te = te_ref[:, :e]
    tn = tn_ref[...]
    s = _lrelu(te + tn)
    s = jnp.where(ht > 0, s, jnp.float32(_NEG))
    m = jnp.max(s, axis=1, keepdims=True)
    p = jnp.exp(s - m)
    p = p / jnp.sum(p, axis=1, keepdims=True)
    node = _dot(p, edge_ref[...])
    elu = jnp.where(node > 0, node, jnp.exp(jnp.minimum(node, 0.0)) - 1.0)
    return elu + xin_ref[...]


def _node_mid_kernel(ht_ref, te_ref, tn_ref, edge_ref, xin_ref,
                     w1_ref, w2_ref, a_ref, a2_ref, q_ref,
                     h_ref, xt_ref, ln_ref, tn2_ref, *, e, d):
    h = _node_body(ht_ref, te_ref, tn_ref, edge_ref, xin_ref, e=e)
    h_ref[...] = h
    x4 = _dot(h, w2_ref[...])
    xt_ref[...] = _dot(h, w1_ref[...])
    sq = _dot(q_ref[...], a_ref[:d, :])
    s = _dot(x4, a_ref[d:, :]) + sq[0, 0]
    ln_ref[...] = _lrelu(s)
    tn2_ref[...] = _dot(x4, a2_ref[:d, :])


def _node_final_kernel(ht_ref, te_ref, tn_ref, edge_ref, xin_ref,
                       out_ref, *, e):
    out_ref[...] = _node_body(ht_ref, te_ref, tn_ref, edge_ref, xin_ref, e=e)


def _node_mid(ht, te, tn, edge, xin, w1, w2, a, a2, q, nb):
    n, e_pad = ht.shape
    e, d = edge.shape
    grid = n // nb
    return pl.pallas_call(
        functools.partial(_node_mid_kernel, e=e, d=d),
        grid=(grid,),
        in_specs=[
            pl.BlockSpec((nb, e_pad), lambda i: (i, 0)),
            pl.BlockSpec((1, e_pad), lambda i: (0, 0)),
            pl.BlockSpec((nb, 1), lambda i: (i, 0)),
            pl.BlockSpec((e, d), lambda i: (0, 0)),
            pl.BlockSpec((nb, d), lambda i: (i, 0)),
            pl.BlockSpec((d, d), lambda i: (0, 0)),
            pl.BlockSpec((d, d), lambda i: (0, 0)),
            pl.BlockSpec((2 * d, 1), lambda i: (0, 0)),
            pl.BlockSpec((2 * d, 1), lambda i: (0, 0)),
            pl.BlockSpec((1, d), lambda i: (0, 0)),
        ],
        out_specs=[
            pl.BlockSpec((nb, d), lambda i: (i, 0)),
            pl.BlockSpec((nb, d), lambda i: (i, 0)),
            pl.BlockSpec((nb, 1), lambda i: (i, 0)),
            pl.BlockSpec((nb, 1), lambda i: (i, 0)),
        ],
        out_shape=[
            jax.ShapeDtypeStruct((n, d), jnp.float32),
            jax.ShapeDtypeStruct((n, d), jnp.float32),
            jax.ShapeDtypeStruct((n, 1), jnp.float32),
            jax.ShapeDtypeStruct((n, 1), jnp.float32),
        ],
    )(ht, te, tn, edge, xin, w1, w2, a, a2, q)


def _node_final(ht, te, tn, edge, xin, nb):
    n, e_pad = ht.shape
    e, d = edge.shape
    grid = n // nb
    return pl.pallas_call(
        functools.partial(_node_final_kernel, e=e),
        grid=(grid,),
        in_specs=[
            pl.BlockSpec((nb, e_pad), lambda i: (i, 0)),
            pl.BlockSpec((1, e_pad), lambda i: (0, 0)),
            pl.BlockSpec((nb, 1), lambda i: (i, 0)),
            pl.BlockSpec((e, d), lambda i: (0, 0)),
            pl.BlockSpec((nb, d), lambda i: (i, 0)),
        ],
        out_specs=pl.BlockSpec((nb, d), lambda i: (i, 0)),
        out_shape=jax.ShapeDtypeStruct((n, d), jnp.float32),
    )(ht, te, tn, edge, xin)


def kernel(x, H, W1_1, W2_1, W3_1, a_1, a2_1, q_1,
           W1_2, W2_2, W3_2, a_2, a2_2, q_2):
    _, n, d = x.shape
    e = H.shape[1]
    x2 = x[0]
    h2 = H[0]
    nb_prep = 2000
    nb_node = 1000
    eb = 128

    xt1, ln1, tn1 = _prep(x2, W1_1, W2_1, a_1, a2_1, q_1, nb_prep)
    edge1, te1, ht = _edge(h2, xt1, ln1, W3_1, a2_1, eb, with_ht=True)
    h, xt2, ln2, tn2 = _node_mid(ht, te1, tn1, edge1, x2,
                                 W1_2, W2_2, a_2, a2_2, q_2, nb_node)
    edge2, te2 = _edge(h2, xt2, ln2, W3_2, a2_2, eb, with_ht=False)
    out = _node_final(ht, te2, tn2, edge2, h, nb_node)
    return out[None]


# speedup vs baseline: 1.7420x; 1.2227x over previous
import functools

import jax
import jax.numpy as jnp
from jax.experimental import pallas as pl
from jax.experimental.pallas import tpu as pltpu

_ALPHA = 0.2
_NEG = -9e15


def _dot(a, b):
    return jnp.dot(a, b, preferred_element_type=jnp.float32)


def _lrelu(x):
    return jnp.where(x > 0, x, _ALPHA * x)


def _edge_kernel(x_ref, h_ref, w1_ref, w2_ref, w3_ref, a_ref, a2_ref, q_ref,
                 edge_ref, te_ref, tn_ref, h8_ref, ht8_ref,
                 wxt_scr, w_scr, *, d, compress):
    i = pl.program_id(0)

    @pl.when(i == 0)
    def _():
        xb = x_ref[...]
        x4 = _dot(xb, w2_ref[...])
        xt = _dot(xb, w1_ref[...])
        sq = _dot(q_ref[...], a_ref[:d, :])
        ln = _lrelu(_dot(x4, a_ref[d:, :]) + sq[0, 0])
        m = jnp.max(ln)
        w = jnp.exp(ln - m)
        w_scr[...] = w
        wxt_scr[...] = xt * w
        tn_ref[...] = _dot(x4, a2_ref[:d, :])

    hb = h_ref[...]
    if compress:
        h8_ref[...] = hb.astype(jnp.int8)
        ht8_ref[...] = hb.T.astype(jnp.int8)
        hbf = hb
    else:
        hbf = hb.astype(jnp.float32)
    num = _dot(hbf, wxt_scr[...])
    z = _dot(hbf, w_scr[...])
    edge = num / z
    edge_ref[...] = edge
    e4 = _dot(edge, w3_ref[...])
    te_ref[...] = jax.lax.dot_general(
        a2_ref[d:, :], e4, (((0,), (1,)), ((), ())),
        preferred_element_type=jnp.float32)


def _edge(x2, h_in, w1, w2, w3, a, a2, q, eb, compress):
    n, d = x2.shape
    e = h_in.shape[0]
    grid = pl.cdiv(e, eb)
    e_pad = grid * eb
    out_specs = [
        pl.BlockSpec((eb, d), lambda i: (i, 0)),
        pl.BlockSpec((1, eb), lambda i: (0, i)),
        pl.BlockSpec((n, 1), lambda i: (0, 0)),
    ]
    out_shape = [
        jax.ShapeDtypeStruct((e, d), jnp.float32),
        jax.ShapeDtypeStruct((1, e_pad), jnp.float32),
        jax.ShapeDtypeStruct((n, 1), jnp.float32),
    ]
    if compress:
        out_specs += [
            pl.BlockSpec((eb, n), lambda i: (i, 0)),
            pl.BlockSpec((n, eb), lambda i: (0, i)),
        ]
        out_shape += [
            jax.ShapeDtypeStruct((e, n), jnp.int8),
            jax.ShapeDtypeStruct((n, e_pad), jnp.int8),
        ]
        body = functools.partial(_edge_kernel, d=d, compress=True)
    else:
        body = functools.partial(_edge_kernel_nc, d=d)
    return pl.pallas_call(
        body,
        grid=(grid,),
        in_specs=[
            pl.BlockSpec((n, d), lambda i: (0, 0)),
            pl.BlockSpec((eb, n), lambda i: (i, 0)),
            pl.BlockSpec((d, d), lambda i: (0, 0)),
            pl.BlockSpec((d, d), lambda i: (0, 0)),
            pl.BlockSpec((d, d), lambda i: (0, 0)),
            pl.BlockSpec((2 * d, 1), lambda i: (0, 0)),
            pl.BlockSpec((2 * d, 1), lambda i: (0, 0)),
            pl.BlockSpec((1, d), lambda i: (0, 0)),
        ],
        out_specs=out_specs,
        out_shape=out_shape,
        scratch_shapes=[
            pltpu.VMEM((n, d), jnp.float32),
            pltpu.VMEM((n, 1), jnp.float32),
        ],
    )(x2, h_in, w1, w2, w3, a, a2, q)


def _edge_kernel_nc(x_ref, h_ref, w1_ref, w2_ref, w3_ref, a_ref, a2_ref,
                    q_ref, edge_ref, te_ref, tn_ref, wxt_scr, w_scr, *, d):
    _edge_kernel(x_ref, h_ref, w1_ref, w2_ref, w3_ref, a_ref, a2_ref, q_ref,
                 edge_ref, te_ref, tn_ref, None, None, wxt_scr, w_scr,
                 d=d, compress=False)


def _node_kernel(ht8_ref, te_ref, tn_ref, edge_ref, xin_ref, out_ref, *, e):
    ht = ht8_ref[:, :e].astype(jnp.float32)
    te = te_ref[:, :e]
    tn = tn_ref[...]
    s = _lrelu(te + tn)
    s = jnp.where(ht > 0, s, jnp.float32(_NEG))
    m = jnp.max(s, axis=1, keepdims=True)
    p = jnp.exp(s - m)
    node = _dot(p, edge_ref[...])
    node = node * (1.0 / jnp.sum(p, axis=1, keepdims=True))
    elu = jnp.where(node > 0, node, jnp.exp(jnp.minimum(node, 0.0)) - 1.0)
    out_ref[...] = elu + xin_ref[...]


def _node(ht8, te, tn, edge, xin, nb):
    n, e_pad = ht8.shape
    e, d = edge.shape
    grid = n // nb
    return pl.pallas_call(
        functools.partial(_node_kernel, e=e),
        grid=(grid,),
        in_specs=[
            pl.BlockSpec((nb, e_pad), lambda i: (i, 0)),
            pl.BlockSpec((1, e_pad), lambda i: (0, 0)),
            pl.BlockSpec((nb, 1), lambda i: (i, 0)),
            pl.BlockSpec((e, d), lambda i: (0, 0)),
            pl.BlockSpec((nb, d), lambda i: (i, 0)),
        ],
        out_specs=pl.BlockSpec((nb, d), lambda i: (i, 0)),
        out_shape=jax.ShapeDtypeStruct((n, d), jnp.float32),
    )(ht8, te, tn, edge, xin)


def kernel(x, H, W1_1, W2_1, W3_1, a_1, a2_1, q_1,
           W1_2, W2_2, W3_2, a_2, a2_2, q_2):
    x2 = x[0]
    h2 = H[0]
    eb = 128
    nb_node = 1000

    edge1, te1, tn1, h8, ht8 = _edge(
        x2, h2, W1_1, W2_1, W3_1, a_1, a2_1, q_1, eb, compress=True)
    h = _node(ht8, te1, tn1, edge1, x2, nb_node)
    edge2, te2, tn2 = _edge(
        h, h8, W1_2, W2_2, W3_2, a_2, a2_2, q_2, eb, compress=False)
    out = _node(ht8, te2, tn2, edge2, h, nb_node)
    return out[None]


# speedup vs baseline: 1.9246x; 1.1048x over previous
import functools

import jax
import jax.numpy as jnp
from jax.experimental import pallas as pl
from jax.experimental.pallas import tpu as pltpu

_ALPHA = 0.2
_NEG = -9e15


def _dot(a, b):
    return jnp.dot(a, b, preferred_element_type=jnp.float32)


def _lrelu(x):
    return jnp.where(x > 0, x, _ALPHA * x)


def _edge_kernel(x_ref, h_ref, w1_ref, w2_ref, w3_ref, a_ref, a2_ref, q_ref,
                 edge_ref, te_ref, tn_ref, h8_ref, ht8_ref,
                 wxt_scr, w_scr, *, d, compress):
    i = pl.program_id(0)

    @pl.when(i == 0)
    def _():
        xb = x_ref[...]
        x4 = _dot(xb, w2_ref[...])
        xt = _dot(xb, w1_ref[...])
        sq = _dot(q_ref[...], a_ref[:d, :])
        ln = _lrelu(_dot(x4, a_ref[d:, :]) + sq[0, 0])
        m = jnp.max(ln)
        w = jnp.exp(ln - m)
        w_scr[...] = w
        wxt_scr[...] = xt * w
        tn_ref[...] = _dot(x4, a2_ref[:d, :])

    hb = h_ref[...]
    if compress:
        h8_ref[...] = hb.astype(jnp.int8)
        ht8_ref[...] = hb.T.astype(jnp.int8)
        hbf = hb
    else:
        hbf = hb.astype(jnp.float32)
    num = _dot(hbf, wxt_scr[...])
    z = _dot(hbf, w_scr[...])
    edge = num / z
    edge_ref[...] = edge
    e4 = _dot(edge, w3_ref[...])
    te_ref[...] = jax.lax.dot_general(
        a2_ref[d:, :], e4, (((0,), (1,)), ((), ())),
        preferred_element_type=jnp.float32)


def _edge(x2, h_in, w1, w2, w3, a, a2, q, eb, compress):
    n, d = x2.shape
    e = h_in.shape[0]
    grid = pl.cdiv(e, eb)
    e_pad = grid * eb
    out_specs = [
        pl.BlockSpec((eb, d), lambda i: (i, 0)),
        pl.BlockSpec((1, eb), lambda i: (0, i)),
        pl.BlockSpec((n, 1), lambda i: (0, 0)),
    ]
    out_shape = [
        jax.ShapeDtypeStruct((e, d), jnp.float32),
        jax.ShapeDtypeStruct((1, e_pad), jnp.float32),
        jax.ShapeDtypeStruct((n, 1), jnp.float32),
    ]
    if compress:
        out_specs += [
            pl.BlockSpec((eb, n), lambda i: (i, 0)),
            pl.BlockSpec((n, eb), lambda i: (0, i)),
        ]
        out_shape += [
            jax.ShapeDtypeStruct((e, n), jnp.int8),
            jax.ShapeDtypeStruct((n, e_pad), jnp.int8),
        ]
        body = functools.partial(_edge_kernel, d=d, compress=True)
    else:
        body = functools.partial(_edge_kernel_nc, d=d)
    return pl.pallas_call(
        body,
        grid=(grid,),
        in_specs=[
            pl.BlockSpec((n, d), lambda i: (0, 0)),
            pl.BlockSpec((eb, n), lambda i: (i, 0)),
            pl.BlockSpec((d, d), lambda i: (0, 0)),
            pl.BlockSpec((d, d), lambda i: (0, 0)),
            pl.BlockSpec((d, d), lambda i: (0, 0)),
            pl.BlockSpec((2 * d, 1), lambda i: (0, 0)),
            pl.BlockSpec((2 * d, 1), lambda i: (0, 0)),
            pl.BlockSpec((1, d), lambda i: (0, 0)),
        ],
        out_specs=out_specs,
        out_shape=out_shape,
        scratch_shapes=[
            pltpu.VMEM((n, d), jnp.float32),
            pltpu.VMEM((n, 1), jnp.float32),
        ],
    )(x2, h_in, w1, w2, w3, a, a2, q)


def _edge_kernel_nc(x_ref, h_ref, w1_ref, w2_ref, w3_ref, a_ref, a2_ref,
                    q_ref, edge_ref, te_ref, tn_ref, wxt_scr, w_scr, *, d):
    _edge_kernel(x_ref, h_ref, w1_ref, w2_ref, w3_ref, a_ref, a2_ref, q_ref,
                 edge_ref, te_ref, tn_ref, None, None, wxt_scr, w_scr,
                 d=d, compress=False)


def _node_kernel(ht8_ref, te_ref, tn_ref, edge_ref, xin_ref, out_ref, *, e):
    ht = ht8_ref[:, :e].astype(jnp.float32)
    te = te_ref[:, :e]
    tn = tn_ref[...]
    s = te + tn
    p = jnp.exp(jnp.maximum(s, _ALPHA * s)) * ht
    node = _dot(p, edge_ref[...])
    node = node * (1.0 / jnp.sum(p, axis=1, keepdims=True))
    elu = jnp.where(node > 0, node, jnp.exp(jnp.minimum(node, 0.0)) - 1.0)
    out_ref[...] = elu + xin_ref[...]


def _node(ht8, te, tn, edge, xin, nb):
    n, e_pad = ht8.shape
    e, d = edge.shape
    grid = n // nb
    return pl.pallas_call(
        functools.partial(_node_kernel, e=e),
        grid=(grid,),
        in_specs=[
            pl.BlockSpec((nb, e_pad), lambda i: (i, 0)),
            pl.BlockSpec((1, e_pad), lambda i: (0, 0)),
            pl.BlockSpec((nb, 1), lambda i: (i, 0)),
            pl.BlockSpec((e, d), lambda i: (0, 0)),
            pl.BlockSpec((nb, d), lambda i: (i, 0)),
        ],
        out_specs=pl.BlockSpec((nb, d), lambda i: (i, 0)),
        out_shape=jax.ShapeDtypeStruct((n, d), jnp.float32),
    )(ht8, te, tn, edge, xin)


def kernel(x, H, W1_1, W2_1, W3_1, a_1, a2_1, q_1,
           W1_2, W2_2, W3_2, a_2, a2_2, q_2):
    x2 = x[0]
    h2 = H[0]
    eb = 128
    nb_node = 1000

    edge1, te1, tn1, h8, ht8 = _edge(
        x2, h2, W1_1, W2_1, W3_1, a_1, a2_1, q_1, eb, compress=True)
    h = _node(ht8, te1, tn1, edge1, x2, nb_node)
    edge2, te2, tn2 = _edge(
        h, h8, W1_2, W2_2, W3_2, a_2, a2_2, q_2, eb, compress=False)
    out = _node(ht8, te2, tn2, edge2, h, nb_node)
    return out[None]


# speedup vs baseline: 2.0058x; 1.0422x over previous
import functools

import jax
import jax.numpy as jnp
from jax.experimental import pallas as pl
from jax.experimental.pallas import tpu as pltpu

_ALPHA = 0.2
_NEG = -9e15


def _dot(a, b):
    return jnp.dot(a, b, preferred_element_type=jnp.float32)


def _dott(a, b):
    return jax.lax.dot_general(a, b, (((0,), (0,)), ((), ())),
                               preferred_element_type=jnp.float32)


def _lrelu(x):
    return jnp.where(x > 0, x, _ALPHA * x)


def _edge1_kernel(x_ref, h_ref, w1_ref, w2_ref, w3_ref, a_ref, a2_ref, q_ref,
                  edge_ref, te_ref, tn_ref, ht8_ref, wxt_scr, w_scr, *, d):
    i = pl.program_id(0)

    @pl.when(i == 0)
    def _():
        xb = x_ref[...]
        x4 = _dot(xb, w2_ref[...])
        xt = _dot(xb, w1_ref[...])
        sq = _dot(q_ref[...], a_ref[:d, :])
        ln = _lrelu(_dot(x4, a_ref[d:, :]) + sq[0, 0])
        m = jnp.max(ln)
        w = jnp.exp(ln - m)
        w_scr[...] = w
        wxt_scr[...] = xt * w
        tn_ref[...] = _dot(x4, a2_ref[:d, :])

    hb = h_ref[...]
    ht8_ref[...] = hb.T.astype(jnp.int8)
    num = _dot(hb, wxt_scr[...])
    z = _dot(hb, w_scr[...])
    edge = num / z
    edge_ref[...] = edge
    e4 = _dot(edge, w3_ref[...])
    te_ref[...] = jax.lax.dot_general(
        a2_ref[d:, :], e4, (((0,), (1,)), ((), ())),
        preferred_element_type=jnp.float32)


def _edge1(x2, h2, w1, w2, w3, a, a2, q, eb):
    e, n = h2.shape
    d = x2.shape[1]
    grid = pl.cdiv(e, eb)
    e_pad = grid * eb
    return pl.pallas_call(
        functools.partial(_edge1_kernel, d=d),
        grid=(grid,),
        in_specs=[
            pl.BlockSpec((n, d), lambda i: (0, 0)),
            pl.BlockSpec((eb, n), lambda i: (i, 0)),
            pl.BlockSpec((d, d), lambda i: (0, 0)),
            pl.BlockSpec((d, d), lambda i: (0, 0)),
            pl.BlockSpec((d, d), lambda i: (0, 0)),
            pl.BlockSpec((2 * d, 1), lambda i: (0, 0)),
            pl.BlockSpec((2 * d, 1), lambda i: (0, 0)),
            pl.BlockSpec((1, d), lambda i: (0, 0)),
        ],
        out_specs=[
            pl.BlockSpec((eb, d), lambda i: (i, 0)),
            pl.BlockSpec((1, eb), lambda i: (0, i)),
            pl.BlockSpec((n, 1), lambda i: (0, 0)),
            pl.BlockSpec((n, eb), lambda i: (0, i)),
        ],
        out_shape=[
            jax.ShapeDtypeStruct((e, d), jnp.float32),
            jax.ShapeDtypeStruct((1, e_pad), jnp.float32),
            jax.ShapeDtypeStruct((n, 1), jnp.float32),
            jax.ShapeDtypeStruct((n, e_pad), jnp.int8),
        ],
        scratch_shapes=[
            pltpu.VMEM((n, d), jnp.float32),
            pltpu.VMEM((n, 1), jnp.float32),
        ],
    )(x2, h2, w1, w2, w3, a, a2, q)


def _node_block(ht, te_ref, tn_ref, edge_ref, xin_ref, e):
    te = te_ref[:, :e]
    tn = tn_ref[...]
    s = te + tn
    p = jnp.exp(jnp.maximum(s, _ALPHA * s)) * ht[:, :e]
    node = _dot(p, edge_ref[...])
    node = node * (1.0 / jnp.sum(p, axis=1, keepdims=True))
    elu = jnp.where(node > 0, node, jnp.exp(jnp.minimum(node, 0.0)) - 1.0)
    return elu + xin_ref[...]


def _mid_kernel(ht8_ref, te_ref, tn_ref, edge_ref, xin_ref,
                w1_ref, w2_ref, w3_ref, a_ref, a2_ref, q_ref,
                h_ref, tn2_ref, edge2_ref, te2_ref,
                numt_scr, z_scr, m_scr, *, e, d, nsteps):
    i = pl.program_id(0)
    htf = ht8_ref[...].astype(jnp.float32)
    h = _node_block(htf, te_ref, tn_ref, edge_ref, xin_ref, e)
    h_ref[...] = h

    x4 = _dot(h, w2_ref[...])
    xt = _dot(h, w1_ref[...])
    sq = _dot(q_ref[...], a_ref[:d, :])
    ln = _lrelu(_dot(x4, a_ref[d:, :]) + sq[0, 0])
    tn2_ref[...] = _dot(x4, a2_ref[:d, :])
    mj = jnp.max(ln)

    @pl.when(i == 0)
    def _():
        m_scr[0, 0] = mj
        w = jnp.exp(ln - mj)
        numt_scr[...] = _dott(xt * w, htf)
        z_scr[...] = _dott(w, htf)

    @pl.when(i > 0)
    def _():
        m_old = m_scr[0, 0]
        m_new = jnp.maximum(m_old, mj)
        m_scr[0, 0] = m_new
        sc = jnp.exp(m_old - m_new)
        w = jnp.exp(ln - m_new)
        numt_scr[...] = numt_scr[...] * sc + _dott(xt * w, htf)
        z_scr[...] = z_scr[...] * sc + _dott(w, htf)

    @pl.when(i == nsteps - 1)
    def _():
        edge2t = numt_scr[...] / z_scr[...]
        edge2_ref[...] = edge2t.T[:e, :]
        e4t = _dott(w3_ref[...], edge2t)
        te2_ref[...] = _dott(a2_ref[d:, :], e4t)


def _mid(ht8, te, tn, edge, xin, w1, w2, w3, a, a2, q, nb):
    n, e_pad = ht8.shape
    e, d = edge.shape
    grid = n // nb
    return pl.pallas_call(
        functools.partial(_mid_kernel, e=e, d=d, nsteps=grid),
        grid=(grid,),
        in_specs=[
            pl.BlockSpec((nb, e_pad), lambda i: (i, 0)),
            pl.BlockSpec((1, e_pad), lambda i: (0, 0)),
            pl.BlockSpec((nb, 1), lambda i: (i, 0)),
            pl.BlockSpec((e, d), lambda i: (0, 0)),
            pl.BlockSpec((nb, d), lambda i: (i, 0)),
            pl.BlockSpec((d, d), lambda i: (0, 0)),
            pl.BlockSpec((d, d), lambda i: (0, 0)),
            pl.BlockSpec((d, d), lambda i: (0, 0)),
            pl.BlockSpec((2 * d, 1), lambda i: (0, 0)),
            pl.BlockSpec((2 * d, 1), lambda i: (0, 0)),
            pl.BlockSpec((1, d), lambda i: (0, 0)),
        ],
        out_specs=[
            pl.BlockSpec((nb, d), lambda i: (i, 0)),
            pl.BlockSpec((nb, 1), lambda i: (i, 0)),
            pl.BlockSpec((e, d), lambda i: (0, 0)),
            pl.BlockSpec((1, e_pad), lambda i: (0, 0)),
        ],
        out_shape=[
            jax.ShapeDtypeStruct((n, d), jnp.float32),
            jax.ShapeDtypeStruct((n, 1), jnp.float32),
            jax.ShapeDtypeStruct((e, d), jnp.float32),
            jax.ShapeDtypeStruct((1, e_pad), jnp.float32),
        ],
        scratch_shapes=[
            pltpu.VMEM((d, e_pad), jnp.float32),
            pltpu.VMEM((1, e_pad), jnp.float32),
            pltpu.SMEM((1, 1), jnp.float32),
        ],
    )(ht8, te, tn, edge, xin, w1, w2, w3, a, a2, q)


def _node_kernel(ht8_ref, te_ref, tn_ref, edge_ref, xin_ref, out_ref, *, e):
    htf = ht8_ref[...].astype(jnp.float32)
    out_ref[...] = _node_block(htf, te_ref, tn_ref, edge_ref, xin_ref, e)


def _node(ht8, te, tn, edge, xin, nb):
    n, e_pad = ht8.shape
    e, d = edge.shape
    grid = n // nb
    return pl.pallas_call(
        functools.partial(_node_kernel, e=e),
        grid=(grid,),
        in_specs=[
            pl.BlockSpec((nb, e_pad), lambda i: (i, 0)),
            pl.BlockSpec((1, e_pad), lambda i: (0, 0)),
            pl.BlockSpec((nb, 1), lambda i: (i, 0)),
            pl.BlockSpec((e, d), lambda i: (0, 0)),
            pl.BlockSpec((nb, d), lambda i: (i, 0)),
        ],
        out_specs=pl.BlockSpec((nb, d), lambda i: (i, 0)),
        out_shape=jax.ShapeDtypeStruct((n, d), jnp.float32),
    )(ht8, te, tn, edge, xin)


def kernel(x, H, W1_1, W2_1, W3_1, a_1, a2_1, q_1,
           W1_2, W2_2, W3_2, a_2, a2_2, q_2):
    x2 = x[0]
    h2 = H[0]
    eb = 128
    nb_node = 1000

    edge1, te1, tn1, ht8 = _edge1(
        x2, h2, W1_1, W2_1, W3_1, a_1, a2_1, q_1, eb)
    h, tn2, edge2, te2 = _mid(
        ht8, te1, tn1, edge1, x2, W1_2, W2_2, W3_2, a_2, a2_2, q_2, nb_node)
    out = _node(ht8, te2, tn2, edge2, h, nb_node)
    return out[None]


# speedup vs baseline: 2.1226x; 1.0583x over previous
import functools

import jax
import jax.numpy as jnp
from jax.experimental import pallas as pl
from jax.experimental.pallas import tpu as pltpu

_ALPHA = 0.2
_NEG = -9e15


def _dot(a, b):
    return jnp.dot(a, b, preferred_element_type=jnp.float32)


def _dott(a, b):
    return jax.lax.dot_general(a, b, (((0,), (0,)), ((), ())),
                               preferred_element_type=jnp.float32)


def _lrelu(x):
    return jnp.where(x > 0, x, _ALPHA * x)


def _edge1_kernel(x_ref, h_ref, w1_ref, w2_ref, w3_ref, a_ref, a2_ref, q_ref,
                  edge_ref, te_ref, tn_ref, ht8_ref, wxt_scr, w_scr, *, d):
    i = pl.program_id(0)

    @pl.when(i == 0)
    def _():
        xb = x_ref[...]
        x4 = _dot(xb, w2_ref[...])
        xt = _dot(xb, w1_ref[...])
        sq = _dot(q_ref[...], a_ref[:d, :])
        ln = _lrelu(_dot(x4, a_ref[d:, :]) + sq[0, 0])
        m = jnp.max(ln)
        w = jnp.exp(ln - m)
        w_scr[...] = w.astype(jnp.bfloat16)
        wxt_scr[...] = (xt * w).astype(jnp.bfloat16)
        tn_ref[...] = _dot(x4, a2_ref[:d, :])

    hb = h_ref[...]
    ht8_ref[...] = hb.T.astype(jnp.int8)
    hb_bf = hb.astype(jnp.bfloat16)
    num = _dot(hb_bf, wxt_scr[...])
    z = _dot(hb_bf, w_scr[...])
    edge = num / z
    edge_ref[...] = edge
    e4 = _dot(edge, w3_ref[...])
    te_ref[...] = jax.lax.dot_general(
        a2_ref[d:, :], e4, (((0,), (1,)), ((), ())),
        preferred_element_type=jnp.float32)


def _edge1(x2, h2, w1, w2, w3, a, a2, q, eb):
    e, n = h2.shape
    d = x2.shape[1]
    grid = pl.cdiv(e, eb)
    e_pad = grid * eb
    return pl.pallas_call(
        functools.partial(_edge1_kernel, d=d),
        grid=(grid,),
        in_specs=[
            pl.BlockSpec((n, d), lambda i: (0, 0)),
            pl.BlockSpec((eb, n), lambda i: (i, 0)),
            pl.BlockSpec((d, d), lambda i: (0, 0)),
            pl.BlockSpec((d, d), lambda i: (0, 0)),
            pl.BlockSpec((d, d), lambda i: (0, 0)),
            pl.BlockSpec((2 * d, 1), lambda i: (0, 0)),
            pl.BlockSpec((2 * d, 1), lambda i: (0, 0)),
            pl.BlockSpec((1, d), lambda i: (0, 0)),
        ],
        out_specs=[
            pl.BlockSpec((eb, d), lambda i: (i, 0)),
            pl.BlockSpec((1, eb), lambda i: (0, i)),
            pl.BlockSpec((n, 1), lambda i: (0, 0)),
            pl.BlockSpec((n, eb), lambda i: (0, i)),
        ],
        out_shape=[
            jax.ShapeDtypeStruct((e, d), jnp.float32),
            jax.ShapeDtypeStruct((1, e_pad), jnp.float32),
            jax.ShapeDtypeStruct((n, 1), jnp.float32),
            jax.ShapeDtypeStruct((n, e_pad), jnp.int8),
        ],
        scratch_shapes=[
            pltpu.VMEM((n, d), jnp.bfloat16),
            pltpu.VMEM((n, 1), jnp.bfloat16),
        ],
    )(x2, h2, w1, w2, w3, a, a2, q)


def _node_block(ht, te_ref, tn_ref, edge_ref, xin_ref, e):
    te = te_ref[:, :e].astype(jnp.bfloat16)
    tn = tn_ref[...].astype(jnp.bfloat16)
    s = te + tn
    p = jnp.exp(jnp.maximum(s, jnp.bfloat16(_ALPHA) * s)) * ht[:, :e]
    edge_bf = edge_ref[...].astype(jnp.bfloat16)
    node = _dot(p, edge_bf)
    z = jnp.sum(p.astype(jnp.float32), axis=1, keepdims=True)
    node = node * (1.0 / z)
    elu = jnp.where(node > 0, node, jnp.exp(jnp.minimum(node, 0.0)) - 1.0)
    return elu + xin_ref[...]


def _mid_kernel(ht8_ref, te_ref, tn_ref, edge_ref, xin_ref,
                w1_ref, w2_ref, w3_ref, a_ref, a2_ref, q_ref,
                h_ref, tn2_ref, edge2_ref, te2_ref,
                numt_scr, z_scr, m_scr, *, e, d, nsteps):
    i = pl.program_id(0)
    htf = ht8_ref[...].astype(jnp.bfloat16)
    h = _node_block(htf, te_ref, tn_ref, edge_ref, xin_ref, e)
    h_ref[...] = h

    x4 = _dot(h, w2_ref[...])
    xt = _dot(h, w1_ref[...])
    sq = _dot(q_ref[...], a_ref[:d, :])
    ln = _lrelu(_dot(x4, a_ref[d:, :]) + sq[0, 0])
    tn2_ref[...] = _dot(x4, a2_ref[:d, :])
    mj = jnp.max(ln)

    @pl.when(i == 0)
    def _():
        m_scr[0, 0] = mj
        w = jnp.exp(ln - mj)
        numt_scr[...] = _dott((xt * w).astype(jnp.bfloat16), htf)
        z_scr[...] = _dott(w.astype(jnp.bfloat16), htf)

    @pl.when(i > 0)
    def _():
        m_old = m_scr[0, 0]
        m_new = jnp.maximum(m_old, mj)
        m_scr[0, 0] = m_new
        sc = jnp.exp(m_old - m_new)
        w = jnp.exp(ln - m_new)
        numt_scr[...] = numt_scr[...] * sc + _dott((xt * w).astype(jnp.bfloat16), htf)
        z_scr[...] = z_scr[...] * sc + _dott(w.astype(jnp.bfloat16), htf)

    @pl.when(i == nsteps - 1)
    def _():
        edge2t = numt_scr[...] / z_scr[...]
        edge2_ref[...] = edge2t.T[:e, :]
        e4t = _dott(w3_ref[...], edge2t)
        te2_ref[...] = _dott(a2_ref[d:, :], e4t)


def _mid(ht8, te, tn, edge, xin, w1, w2, w3, a, a2, q, nb):
    n, e_pad = ht8.shape
    e, d = edge.shape
    grid = n // nb
    return pl.pallas_call(
        functools.partial(_mid_kernel, e=e, d=d, nsteps=grid),
        grid=(grid,),
        in_specs=[
            pl.BlockSpec((nb, e_pad), lambda i: (i, 0)),
            pl.BlockSpec((1, e_pad), lambda i: (0, 0)),
            pl.BlockSpec((nb, 1), lambda i: (i, 0)),
            pl.BlockSpec((e, d), lambda i: (0, 0)),
            pl.BlockSpec((nb, d), lambda i: (i, 0)),
            pl.BlockSpec((d, d), lambda i: (0, 0)),
            pl.BlockSpec((d, d), lambda i: (0, 0)),
            pl.BlockSpec((d, d), lambda i: (0, 0)),
            pl.BlockSpec((2 * d, 1), lambda i: (0, 0)),
            pl.BlockSpec((2 * d, 1), lambda i: (0, 0)),
            pl.BlockSpec((1, d), lambda i: (0, 0)),
        ],
        out_specs=[
            pl.BlockSpec((nb, d), lambda i: (i, 0)),
            pl.BlockSpec((nb, 1), lambda i: (i, 0)),
            pl.BlockSpec((e, d), lambda i: (0, 0)),
            pl.BlockSpec((1, e_pad), lambda i: (0, 0)),
        ],
        out_shape=[
            jax.ShapeDtypeStruct((n, d), jnp.float32),
            jax.ShapeDtypeStruct((n, 1), jnp.float32),
            jax.ShapeDtypeStruct((e, d), jnp.float32),
            jax.ShapeDtypeStruct((1, e_pad), jnp.float32),
        ],
        scratch_shapes=[
            pltpu.VMEM((d, e_pad), jnp.float32),
            pltpu.VMEM((1, e_pad), jnp.float32),
            pltpu.SMEM((1, 1), jnp.float32),
        ],
    )(ht8, te, tn, edge, xin, w1, w2, w3, a, a2, q)


def _node_kernel(ht8_ref, te_ref, tn_ref, edge_ref, xin_ref, out_ref, *, e):
    htf = ht8_ref[...].astype(jnp.bfloat16)
    out_ref[...] = _node_block(htf, te_ref, tn_ref, edge_ref, xin_ref, e)


def _node(ht8, te, tn, edge, xin, nb):
    n, e_pad = ht8.shape
    e, d = edge.shape
    grid = n // nb
    return pl.pallas_call(
        functools.partial(_node_kernel, e=e),
        grid=(grid,),
        in_specs=[
            pl.BlockSpec((nb, e_pad), lambda i: (i, 0)),
            pl.BlockSpec((1, e_pad), lambda i: (0, 0)),
            pl.BlockSpec((nb, 1), lambda i: (i, 0)),
            pl.BlockSpec((e, d), lambda i: (0, 0)),
            pl.BlockSpec((nb, d), lambda i: (i, 0)),
        ],
        out_specs=pl.BlockSpec((nb, d), lambda i: (i, 0)),
        out_shape=jax.ShapeDtypeStruct((n, d), jnp.float32),
    )(ht8, te, tn, edge, xin)


def kernel(x, H, W1_1, W2_1, W3_1, a_1, a2_1, q_1,
           W1_2, W2_2, W3_2, a_2, a2_2, q_2):
    x2 = x[0]
    h2 = H[0]
    eb = 128
    nb_node = 1000

    edge1, te1, tn1, ht8 = _edge1(
        x2, h2, W1_1, W2_1, W3_1, a_1, a2_1, q_1, eb)
    h, tn2, edge2, te2 = _mid(
        ht8, te1, tn1, edge1, x2, W1_2, W2_2, W3_2, a_2, a2_2, q_2, nb_node)
    out = _node(ht8, te2, tn2, edge2, h, nb_node)
    return out[None]


# speedup vs baseline: 2.3240x; 1.0949x over previous
import functools

import jax
import jax.numpy as jnp
from jax.experimental import pallas as pl
from jax.experimental.pallas import tpu as pltpu

_ALPHA = 0.2
_NEG = -9e15


def _dot(a, b):
    return jnp.dot(a, b, preferred_element_type=jnp.float32)


def _dott(a, b):
    return jax.lax.dot_general(a, b, (((0,), (0,)), ((), ())),
                               preferred_element_type=jnp.float32)


def _lrelu(x):
    return jnp.where(x > 0, x, _ALPHA * x)


def _edge1_kernel(x_ref, h_ref, w1_ref, w2_ref, w3_ref, a_ref, a2_ref, q_ref,
                  edge_ref, te_ref, tn_ref, ht8_ref, wxt_scr, w_scr, *, d):
    i = pl.program_id(0)

    @pl.when(i == 0)
    def _():
        xb = x_ref[...]
        x4 = _dot(xb, w2_ref[...])
        xt = _dot(xb, w1_ref[...])
        sq = _dot(q_ref[...], a_ref[:d, :])
        ln = _lrelu(_dot(x4, a_ref[d:, :]) + sq[0, 0])
        m = jnp.max(ln)
        w = jnp.exp(ln - m)
        w_scr[...] = w.astype(jnp.bfloat16)
        wxt_scr[...] = (xt * w).astype(jnp.bfloat16)
        tn_ref[...] = _dot(x4, a2_ref[:d, :])

    hb = h_ref[...]
    ht8_ref[...] = hb.T.astype(jnp.int8)
    hb_bf = hb.astype(jnp.bfloat16)
    num = _dot(hb_bf, wxt_scr[...])
    z = _dot(hb_bf, w_scr[...])
    edge = num / z
    edge_ref[...] = edge
    e4 = _dot(edge, w3_ref[...])
    te_ref[...] = jax.lax.dot_general(
        a2_ref[d:, :], e4, (((0,), (1,)), ((), ())),
        preferred_element_type=jnp.float32)


def _edge1(x2, h2, w1, w2, w3, a, a2, q, eb):
    e, n = h2.shape
    d = x2.shape[1]
    grid = pl.cdiv(e, eb)
    e_pad = grid * eb
    return pl.pallas_call(
        functools.partial(_edge1_kernel, d=d),
        grid=(grid,),
        in_specs=[
            pl.BlockSpec((n, d), lambda i: (0, 0)),
            pl.BlockSpec((eb, n), lambda i: (i, 0)),
            pl.BlockSpec((d, d), lambda i: (0, 0)),
            pl.BlockSpec((d, d), lambda i: (0, 0)),
            pl.BlockSpec((d, d), lambda i: (0, 0)),
            pl.BlockSpec((2 * d, 1), lambda i: (0, 0)),
            pl.BlockSpec((2 * d, 1), lambda i: (0, 0)),
            pl.BlockSpec((1, d), lambda i: (0, 0)),
        ],
        out_specs=[
            pl.BlockSpec((eb, d), lambda i: (i, 0)),
            pl.BlockSpec((1, eb), lambda i: (0, i)),
            pl.BlockSpec((n, 1), lambda i: (0, 0)),
            pl.BlockSpec((n, eb), lambda i: (0, i)),
        ],
        out_shape=[
            jax.ShapeDtypeStruct((e, d), jnp.float32),
            jax.ShapeDtypeStruct((1, e_pad), jnp.float32),
            jax.ShapeDtypeStruct((n, 1), jnp.float32),
            jax.ShapeDtypeStruct((n, e_pad), jnp.int8),
        ],
        scratch_shapes=[
            pltpu.VMEM((n, d), jnp.bfloat16),
            pltpu.VMEM((n, 1), jnp.bfloat16),
        ],
    )(x2, h2, w1, w2, w3, a, a2, q)


def _node_block(ht, te_ref, tn_ref, edge_ref, xin_ref, e):
    te = te_ref[:, :e].astype(jnp.bfloat16)
    tn = tn_ref[...].astype(jnp.bfloat16)
    s = te + tn
    p = jnp.exp(jnp.maximum(s, jnp.bfloat16(_ALPHA) * s)) * ht[:, :e]
    edge_bf = edge_ref[...].astype(jnp.bfloat16)
    node = _dot(p, edge_bf)
    z = jnp.sum(p.astype(jnp.float32), axis=1, keepdims=True)
    node = node * (1.0 / z)
    elu = jnp.where(node > 0, node, jnp.exp(jnp.minimum(node, 0.0)) - 1.0)
    return elu + xin_ref[...]


def _mid_kernel(ht8_ref, te_ref, tn_ref, edge_ref, xin_ref,
                w1_ref, w2_ref, w3_ref, a_ref, a2_ref, q_ref,
                h_ref, tn2_ref, edge2_ref, te2_ref,
                numt_scr, z_scr, m_scr, *, e, d, nsteps):
    i = pl.program_id(0)
    htf = ht8_ref[...].astype(jnp.bfloat16)
    h = _node_block(htf, te_ref, tn_ref, edge_ref, xin_ref, e)
    h_ref[...] = h

    x4 = _dot(h, w2_ref[...])
    xt = _dot(h, w1_ref[...])
    sq = _dot(q_ref[...], a_ref[:d, :])
    ln = _lrelu(_dot(x4, a_ref[d:, :]) + sq[0, 0])
    tn2_ref[...] = _dot(x4, a2_ref[:d, :])
    mj = jnp.max(ln)

    @pl.when(i == 0)
    def _():
        m_scr[0, 0] = mj
        w = jnp.exp(ln - mj)
        numt_scr[...] = _dott((xt * w).astype(jnp.bfloat16), htf)
        z_scr[...] = _dott(w.astype(jnp.bfloat16), htf)

    @pl.when(i > 0)
    def _():
        m_old = m_scr[0, 0]
        m_new = jnp.maximum(m_old, mj)
        m_scr[0, 0] = m_new
        sc = jnp.exp(m_old - m_new)
        w = jnp.exp(ln - m_new)
        numt_scr[...] = numt_scr[...] * sc + _dott((xt * w).astype(jnp.bfloat16), htf)
        z_scr[...] = z_scr[...] * sc + _dott(w.astype(jnp.bfloat16), htf)

    @pl.when(i == nsteps - 1)
    def _():
        edge2t = numt_scr[...] / z_scr[...]
        edge2_ref[...] = edge2t.T[:e, :]
        e4t = _dott(w3_ref[...], edge2t)
        te2_ref[...] = _dott(a2_ref[d:, :], e4t)


def _mid(ht8, te, tn, edge, xin, w1, w2, w3, a, a2, q, nb):
    n, e_pad = ht8.shape
    e, d = edge.shape
    grid = n // nb
    return pl.pallas_call(
        functools.partial(_mid_kernel, e=e, d=d, nsteps=grid),
        grid=(grid,),
        in_specs=[
            pl.BlockSpec((nb, e_pad), lambda i: (i, 0)),
            pl.BlockSpec((1, e_pad), lambda i: (0, 0)),
            pl.BlockSpec((nb, 1), lambda i: (i, 0)),
            pl.BlockSpec((e, d), lambda i: (0, 0)),
            pl.BlockSpec((nb, d), lambda i: (i, 0)),
            pl.BlockSpec((d, d), lambda i: (0, 0)),
            pl.BlockSpec((d, d), lambda i: (0, 0)),
            pl.BlockSpec((d, d), lambda i: (0, 0)),
            pl.BlockSpec((2 * d, 1), lambda i: (0, 0)),
            pl.BlockSpec((2 * d, 1), lambda i: (0, 0)),
            pl.BlockSpec((1, d), lambda i: (0, 0)),
        ],
        out_specs=[
            pl.BlockSpec((nb, d), lambda i: (i, 0)),
            pl.BlockSpec((nb, 1), lambda i: (i, 0)),
            pl.BlockSpec((e, d), lambda i: (0, 0)),
            pl.BlockSpec((1, e_pad), lambda i: (0, 0)),
        ],
        out_shape=[
            jax.ShapeDtypeStruct((n, d), jnp.float32),
            jax.ShapeDtypeStruct((n, 1), jnp.float32),
            jax.ShapeDtypeStruct((e, d), jnp.float32),
            jax.ShapeDtypeStruct((1, e_pad), jnp.float32),
        ],
        scratch_shapes=[
            pltpu.VMEM((d, e_pad), jnp.float32),
            pltpu.VMEM((1, e_pad), jnp.float32),
            pltpu.SMEM((1, 1), jnp.float32),
        ],
    )(ht8, te, tn, edge, xin, w1, w2, w3, a, a2, q)


def _node_kernel(ht8_ref, te_ref, tn_ref, edge_ref, xin_ref, out_ref, *, e):
    htf = ht8_ref[...].astype(jnp.bfloat16)
    out_ref[...] = _node_block(htf, te_ref, tn_ref, edge_ref, xin_ref, e)


def _node(ht8, te, tn, edge, xin, nb):
    n, e_pad = ht8.shape
    e, d = edge.shape
    grid = n // nb
    return pl.pallas_call(
        functools.partial(_node_kernel, e=e),
        grid=(grid,),
        in_specs=[
            pl.BlockSpec((nb, e_pad), lambda i: (i, 0)),
            pl.BlockSpec((1, e_pad), lambda i: (0, 0)),
            pl.BlockSpec((nb, 1), lambda i: (i, 0)),
            pl.BlockSpec((e, d), lambda i: (0, 0)),
            pl.BlockSpec((nb, d), lambda i: (i, 0)),
        ],
        out_specs=pl.BlockSpec((nb, d), lambda i: (i, 0)),
        out_shape=jax.ShapeDtypeStruct((n, d), jnp.float32),
    )(ht8, te, tn, edge, xin)


def kernel(x, H, W1_1, W2_1, W3_1, a_1, a2_1, q_1,
           W1_2, W2_2, W3_2, a_2, a2_2, q_2):
    x2 = x[0]
    h2 = H[0]
    eb = 256
    nb_node = 2000

    edge1, te1, tn1, ht8 = _edge1(
        x2, h2, W1_1, W2_1, W3_1, a_1, a2_1, q_1, eb)
    h, tn2, edge2, te2 = _mid(
        ht8, te1, tn1, edge1, x2, W1_2, W2_2, W3_2, a_2, a2_2, q_2, nb_node)
    out = _node(ht8, te2, tn2, edge2, h, nb_node)
    return out[None]
